# initial kernel scaffold (unmeasured)
import jax
import jax.numpy as jnp
from jax import lax
from jax.experimental import pallas as pl
from jax.experimental.pallas import tpu as pltpu

N_DEV = 4
NBLK = 1024


def kernel(x, w_mat, scale_x, scale_w):
    m_tot, k_loc = x.shape
    _, n_tot = w_mat.shape
    m_per = m_tot // N_DEV
    n_blocks = n_tot // NBLK

    def body(x_ref, w_ref, sx_ref, sw_ref, out_ref, comm, send_sems, recv_sems):
        nb = pl.program_id(0)
        my = lax.axis_index("i")
        left = lax.rem(my + N_DEV - 1, N_DEV)
        right = lax.rem(my + 1, N_DEV)

        @pl.when(nb == 0)
        def _():
            barrier = pltpu.get_barrier_semaphore()
            for nbr in (left, right):
                pl.semaphore_signal(
                    barrier, inc=1, device_id=(nbr,),
                    device_id_type=pl.DeviceIdType.MESH,
                )
            pl.semaphore_wait(barrier, 2)

        def partial_for(c):
            xs = x_ref[pl.ds(c * m_per, m_per), :]
            return lax.dot_general(
                xs, w_ref[...], (((1,), (0,)), ((), ())),
                preferred_element_type=jnp.float32,
            )

        comm[0, :, :] = partial_for(lax.rem(my + N_DEV - 1, N_DEV))

        for s in range(N_DEV - 1):
            rdma = pltpu.make_async_remote_copy(
                src_ref=comm.at[s],
                dst_ref=comm.at[s + 1],
                send_sem=send_sems.at[s],
                recv_sem=recv_sems.at[s],
                device_id=(right,),
                device_id_type=pl.DeviceIdType.MESH,
            )
            rdma.start()
            rdma.wait()
            c = lax.rem(my + 2 * N_DEV - 2 - s, N_DEV)
            comm[s + 1, :, :] = comm[s + 1, :, :] + partial_for(c)

        scale = sx_ref[0] * sw_ref[0]
        y = comm[N_DEV - 1, :, :] * scale
        out_ref[...] = y / (1.0 + jnp.exp(-jnp.clip(y, -60.0, 60.0)))

    return pl.pallas_call(
        body,
        grid=(n_blocks,),
        in_specs=[
            pl.BlockSpec((m_tot, k_loc), lambda nb: (0, 0)),
            pl.BlockSpec((k_loc, NBLK), lambda nb: (0, nb)),
            pl.BlockSpec(memory_space=pltpu.SMEM),
            pl.BlockSpec(memory_space=pltpu.SMEM),
        ],
        out_specs=pl.BlockSpec((m_per, NBLK), lambda nb: (0, nb)),
        out_shape=jax.ShapeDtypeStruct((m_per, n_tot), jnp.float32),
        scratch_shapes=[
            pltpu.VMEM((N_DEV, m_per, NBLK), jnp.float32),
            pltpu.SemaphoreType.DMA((N_DEV - 1,)),
            pltpu.SemaphoreType.DMA((N_DEV - 1,)),
        ],
        compiler_params=pltpu.CompilerParams(
            dimension_semantics=("arbitrary",),
            collective_id=0,
        ),
    )(x, w_mat, scale_x, scale_w)


# baseline (device time: 1225186 ns/iter reference)
import jax
import jax.numpy as jnp
from jax import lax
from jax.experimental import pallas as pl
from jax.experimental.pallas import tpu as pltpu

N_DEV = 4
NBLK = 1024


def kernel(x, w_mat, scale_x, scale_w):
    m_tot, k_loc = x.shape
    _, n_tot = w_mat.shape
    m_per = m_tot // N_DEV
    n_blocks = n_tot // NBLK

    x = x.astype(jnp.float8_e4m3fn)
    w_mat = w_mat.astype(jnp.float8_e4m3fn)

    def body(x_ref, w_ref, sx_ref, sw_ref, out_ref, comm, send_sems, recv_sems):
        nb = pl.program_id(0)
        my = lax.axis_index("i")
        left = lax.rem(my + N_DEV - 1, N_DEV)
        right = lax.rem(my + 1, N_DEV)

        @pl.when(nb == 0)
        def _():
            barrier = pltpu.get_barrier_semaphore()
            for nbr in (left, right):
                pl.semaphore_signal(
                    barrier, inc=1, device_id=(nbr,),
                    device_id_type=pl.DeviceIdType.MESH,
                )
            pl.semaphore_wait(barrier, 2)

        def partial_for(c):
            xs = x_ref[pl.ds(c * m_per, m_per), :]
            return lax.dot_general(
                xs, w_ref[...], (((1,), (0,)), ((), ())),
                preferred_element_type=jnp.float32,
            )

        comm[0, :, :] = partial_for(lax.rem(my + N_DEV - 1, N_DEV))

        for s in range(N_DEV - 1):
            rdma = pltpu.make_async_remote_copy(
                src_ref=comm.at[s],
                dst_ref=comm.at[s + 1],
                send_sem=send_sems.at[s],
                recv_sem=recv_sems.at[s],
                device_id=(right,),
                device_id_type=pl.DeviceIdType.MESH,
            )
            rdma.start()
            rdma.wait()
            c = lax.rem(my + 2 * N_DEV - 2 - s, N_DEV)
            comm[s + 1, :, :] = comm[s + 1, :, :] + partial_for(c)

        scale = sx_ref[0] * sw_ref[0]
        y = comm[N_DEV - 1, :, :] * scale
        out_ref[...] = y / (1.0 + jnp.exp(-jnp.clip(y, -60.0, 60.0)))

    return pl.pallas_call(
        body,
        grid=(n_blocks,),
        in_specs=[
            pl.BlockSpec((m_tot, k_loc), lambda nb: (0, 0)),
            pl.BlockSpec((k_loc, NBLK), lambda nb: (0, nb)),
            pl.BlockSpec(memory_space=pltpu.SMEM),
            pl.BlockSpec(memory_space=pltpu.SMEM),
        ],
        out_specs=pl.BlockSpec((m_per, NBLK), lambda nb: (0, nb)),
        out_shape=jax.ShapeDtypeStruct((m_per, n_tot), jnp.float32),
        scratch_shapes=[
            pltpu.VMEM((N_DEV, m_per, NBLK), jnp.float32),
            pltpu.SemaphoreType.DMA((N_DEV - 1,)),
            pltpu.SemaphoreType.DMA((N_DEV - 1,)),
        ],
        compiler_params=pltpu.CompilerParams(
            dimension_semantics=("arbitrary",),
            collective_id=0,
            vmem_limit_bytes=50 * 1024 * 1024,
        ),
    )(x, w_mat, scale_x, scale_w)


# device time: 379039 ns/iter; 3.2323x vs baseline; 3.2323x over previous
import jax
import jax.numpy as jnp
from jax import lax
from jax.experimental import pallas as pl
from jax.experimental.pallas import tpu as pltpu

N_DEV = 4
NBLK = 2048
NH = NBLK // 2


def kernel(x, w_mat, scale_x, scale_w):
    m_tot, k_loc = x.shape
    _, n_tot = w_mat.shape
    m_per = m_tot // N_DEV
    n_blocks = n_tot // NBLK

    x = x.astype(jnp.float8_e4m3fn)
    w_mat = w_mat.astype(jnp.float8_e4m3fn)

    def body(x_ref, w_ref, sx_ref, sw_ref, out_ref,
             comm_r, comm_l, send_r, recv_r, send_l, recv_l):
        nb = pl.program_id(0)
        my = lax.axis_index("i")
        left = lax.rem(my + N_DEV - 1, N_DEV)
        right = lax.rem(my + 1, N_DEV)

        @pl.when(nb == 0)
        def _():
            barrier = pltpu.get_barrier_semaphore()
            for nbr in (left, right):
                pl.semaphore_signal(
                    barrier, inc=1, device_id=(nbr,),
                    device_id_type=pl.DeviceIdType.MESH,
                )
            pl.semaphore_wait(barrier, 2)

        def partial_for(c, col0):
            xs = x_ref[pl.ds(c * m_per, m_per), :]
            return lax.dot_general(
                xs, w_ref[:, pl.ds(col0, NH)], (((1,), (0,)), ((), ())),
                preferred_element_type=jnp.float32,
            )

        comm_r[0, :, :] = partial_for(
            lax.rem(my + N_DEV - 1, N_DEV), 0).astype(jnp.bfloat16)
        comm_l[0, :, :] = partial_for(
            lax.rem(my + 1, N_DEV), NH).astype(jnp.bfloat16)

        for s in range(N_DEV - 1):
            rdma_r = pltpu.make_async_remote_copy(
                src_ref=comm_r.at[s], dst_ref=comm_r.at[s + 1],
                send_sem=send_r.at[s], recv_sem=recv_r.at[s],
                device_id=(right,), device_id_type=pl.DeviceIdType.MESH,
            )
            rdma_l = pltpu.make_async_remote_copy(
                src_ref=comm_l.at[s], dst_ref=comm_l.at[s + 1],
                send_sem=send_l.at[s], recv_sem=recv_l.at[s],
                device_id=(left,), device_id_type=pl.DeviceIdType.MESH,
            )
            rdma_r.start()
            rdma_l.start()
            cr = lax.rem(my + 2 * N_DEV - 2 - s, N_DEV)
            cl = lax.rem(my + 2 + s, N_DEV)
            p_r = partial_for(cr, 0)
            p_l = partial_for(cl, NH)
            rdma_r.wait()
            rdma_l.wait()
            if s < N_DEV - 2:
                comm_r[s + 1, :, :] = (
                    comm_r[s + 1, :, :].astype(jnp.float32) + p_r
                ).astype(jnp.bfloat16)
                comm_l[s + 1, :, :] = (
                    comm_l[s + 1, :, :].astype(jnp.float32) + p_l
                ).astype(jnp.bfloat16)
            else:
                scale = sx_ref[0] * sw_ref[0]
                y_r = (comm_r[s + 1, :, :].astype(jnp.float32) + p_r) * scale
                y_l = (comm_l[s + 1, :, :].astype(jnp.float32) + p_l) * scale
                out_ref[:, 0:NH] = y_r / (
                    1.0 + jnp.exp(-jnp.clip(y_r, -60.0, 60.0)))
                out_ref[:, NH:NBLK] = y_l / (
                    1.0 + jnp.exp(-jnp.clip(y_l, -60.0, 60.0)))

    return pl.pallas_call(
        body,
        grid=(n_blocks,),
        in_specs=[
            pl.BlockSpec((m_tot, k_loc), lambda nb: (0, 0)),
            pl.BlockSpec((k_loc, NBLK), lambda nb: (0, nb)),
            pl.BlockSpec(memory_space=pltpu.SMEM),
            pl.BlockSpec(memory_space=pltpu.SMEM),
        ],
        out_specs=pl.BlockSpec((m_per, NBLK), lambda nb: (0, nb)),
        out_shape=jax.ShapeDtypeStruct((m_per, n_tot), jnp.float32),
        scratch_shapes=[
            pltpu.VMEM((N_DEV, m_per, NH), jnp.bfloat16),
            pltpu.VMEM((N_DEV, m_per, NH), jnp.bfloat16),
            pltpu.SemaphoreType.DMA((N_DEV - 1,)),
            pltpu.SemaphoreType.DMA((N_DEV - 1,)),
            pltpu.SemaphoreType.DMA((N_DEV - 1,)),
            pltpu.SemaphoreType.DMA((N_DEV - 1,)),
        ],
        compiler_params=pltpu.CompilerParams(
            dimension_semantics=("arbitrary",),
            collective_id=0,
            vmem_limit_bytes=56 * 1024 * 1024,
        ),
    )(x, w_mat, scale_x, scale_w)


# device time: 245546 ns/iter; 4.9896x vs baseline; 1.5437x over previous
import jax
import jax.numpy as jnp
from jax import lax
from jax.experimental import pallas as pl
from jax.experimental.pallas import tpu as pltpu

N_DEV = 4
NBLK2 = 1024


def _exchange(x, w_mat):
    m_tot, k_loc = x.shape
    _, n_tot = w_mat.shape
    m_per = m_tot // N_DEV
    k_half = k_loc // 2

    def body(x_ref, w_ref, w_full, x_own,
             wrecv_l, wrecv_r, wdiag, send_sems, recv_sems, copy_sems):
        my = lax.axis_index("i")
        left = lax.rem(my + N_DEV - 1, N_DEV)
        right = lax.rem(my + 1, N_DEV)
        diag = lax.rem(my + 2, N_DEV)

        barrier = pltpu.get_barrier_semaphore()
        for nbr in (left, right, diag):
            pl.semaphore_signal(
                barrier, inc=1, device_id=(nbr,),
                device_id_type=pl.DeviceIdType.MESH,
            )
        pl.semaphore_wait(barrier, 3)

        def rc(i, src, dst, dev):
            return pltpu.make_async_remote_copy(
                src_ref=src, dst_ref=dst,
                send_sem=send_sems.at[i], recv_sem=recv_sems.at[i],
                device_id=(dev,), device_id_type=pl.DeviceIdType.MESH,
            )

        a = rc(0, w_ref, wrecv_l, right)
        b = rc(1, w_ref, wrecv_r, left)
        c = rc(2, x_ref.at[pl.ds(right * m_per, m_per), :],
               x_own.at[:, pl.ds(my * k_loc, k_loc)], right)
        d = rc(3, x_ref.at[pl.ds(left * m_per, m_per), :],
               x_own.at[:, pl.ds(my * k_loc, k_loc)], left)
        e = rc(4, x_ref.at[pl.ds(diag * m_per, m_per), :],
               x_own.at[:, pl.ds(my * k_loc, k_loc)], diag)
        for r in (a, b, c, d, e):
            r.start()

        x_own[:, pl.ds(my * k_loc, k_loc)] = x_ref[pl.ds(my * m_per, m_per), :]
        own_cp = pltpu.make_async_copy(
            w_ref, w_full.at[pl.ds(my * k_loc, k_loc), :], copy_sems.at[0])
        own_cp.start()

        a.wait()
        b.wait()
        f = rc(5, wrecv_l.at[pl.ds(0, k_half), :],
               wdiag.at[pl.ds(0, k_half), :], right)
        g = rc(6, wrecv_r.at[pl.ds(k_half, k_half), :],
               wdiag.at[pl.ds(k_half, k_half), :], left)
        f.start()
        g.start()

        cp_l = pltpu.make_async_copy(
            wrecv_l, w_full.at[pl.ds(left * k_loc, k_loc), :], copy_sems.at[1])
        cp_r = pltpu.make_async_copy(
            wrecv_r, w_full.at[pl.ds(right * k_loc, k_loc), :], copy_sems.at[2])
        cp_l.start()
        cp_r.start()

        f.wait()
        g.wait()
        cp_d = pltpu.make_async_copy(
            wdiag, w_full.at[pl.ds(diag * k_loc, k_loc), :], copy_sems.at[3])
        cp_d.start()

        c.wait()
        d.wait()
        e.wait()
        own_cp.wait()
        cp_l.wait()
        cp_r.wait()
        cp_d.wait()

    return pl.pallas_call(
        body,
        in_specs=[
            pl.BlockSpec(memory_space=pltpu.VMEM),
            pl.BlockSpec(memory_space=pltpu.VMEM),
        ],
        out_specs=[
            pl.BlockSpec(memory_space=pl.ANY),
            pl.BlockSpec(memory_space=pltpu.VMEM),
        ],
        out_shape=[
            jax.ShapeDtypeStruct((N_DEV * k_loc, n_tot), jnp.float8_e4m3fn),
            jax.ShapeDtypeStruct((m_per, N_DEV * k_loc), jnp.float8_e4m3fn),
        ],
        scratch_shapes=[
            pltpu.VMEM((k_loc, n_tot), jnp.float8_e4m3fn),
            pltpu.VMEM((k_loc, n_tot), jnp.float8_e4m3fn),
            pltpu.VMEM((k_loc, n_tot), jnp.float8_e4m3fn),
            pltpu.SemaphoreType.DMA((7,)),
            pltpu.SemaphoreType.DMA((7,)),
            pltpu.SemaphoreType.DMA((4,)),
        ],
        compiler_params=pltpu.CompilerParams(
            collective_id=0,
            vmem_limit_bytes=56 * 1024 * 1024,
        ),
    )(x, w_mat)


def _gemm_epilogue(x_own, w_full, scale_x, scale_w):
    m_per, k_tot = x_own.shape
    _, n_tot = w_full.shape
    n_blocks = n_tot // NBLK2

    def body(x_ref, w_ref, sx_ref, sw_ref, out_ref):
        acc = lax.dot_general(
            x_ref[...], w_ref[...], (((1,), (0,)), ((), ())),
            preferred_element_type=jnp.float32,
        )
        y = acc * (sx_ref[0] * sw_ref[0])
        out_ref[...] = y / (1.0 + jnp.exp(-jnp.clip(y, -60.0, 60.0)))

    return pl.pallas_call(
        body,
        grid=(n_blocks,),
        in_specs=[
            pl.BlockSpec((m_per, k_tot), lambda nb: (0, 0)),
            pl.BlockSpec((k_tot, NBLK2), lambda nb: (0, nb)),
            pl.BlockSpec(memory_space=pltpu.SMEM),
            pl.BlockSpec(memory_space=pltpu.SMEM),
        ],
        out_specs=pl.BlockSpec((m_per, NBLK2), lambda nb: (0, nb)),
        out_shape=jax.ShapeDtypeStruct((m_per, n_tot), jnp.float32),
        compiler_params=pltpu.CompilerParams(
            dimension_semantics=("arbitrary",),
            vmem_limit_bytes=48 * 1024 * 1024,
        ),
    )(x_own, w_full, scale_x, scale_w)


def kernel(x, w_mat, scale_x, scale_w):
    x = x.astype(jnp.float8_e4m3fn)
    w_mat = w_mat.astype(jnp.float8_e4m3fn)
    w_full, x_own = _exchange(x, w_mat)
    return _gemm_epilogue(x_own, w_full, scale_x, scale_w)


# device time: 236695 ns/iter; 5.1762x vs baseline; 1.0374x over previous
import jax
import jax.numpy as jnp
from jax import lax
from jax.experimental import pallas as pl
from jax.experimental.pallas import tpu as pltpu

N_DEV = 4


def kernel(x, w_mat, scale_x, scale_w):
    m_tot, k_loc = x.shape
    _, n_tot = w_mat.shape
    m_per = m_tot // N_DEV
    k_half = k_loc // 2
    n_half = n_tot // 2

    x = x.astype(jnp.float8_e4m3fn)
    w_mat = w_mat.astype(jnp.float8_e4m3fn)

    def body(x_ref, w_ref, sx_ref, sw_ref, out_ref,
             xg, wl, wr, wd, acc, send_sems, recv_sems, copy_sems):
        my = lax.axis_index("i")
        left = lax.rem(my + N_DEV - 1, N_DEV)
        right = lax.rem(my + 1, N_DEV)
        diag = lax.rem(my + 2, N_DEV)

        barrier = pltpu.get_barrier_semaphore()
        for nbr in (left, right, diag):
            pl.semaphore_signal(
                barrier, inc=1, device_id=(nbr,),
                device_id_type=pl.DeviceIdType.MESH,
            )
        pl.semaphore_wait(barrier, 3)

        def rc(i, src, dst, dev):
            return pltpu.make_async_remote_copy(
                src_ref=src, dst_ref=dst,
                send_sem=send_sems.at[i], recv_sem=recv_sems.at[i],
                device_id=(dev,), device_id_type=pl.DeviceIdType.MESH,
            )

        c = rc(0, x_ref.at[pl.ds(right * m_per, m_per), :], xg.at[my], right)
        d = rc(1, x_ref.at[pl.ds(left * m_per, m_per), :], xg.at[my], left)
        e = rc(2, x_ref.at[pl.ds(diag * m_per, m_per), :], xg.at[my], diag)
        a0 = rc(3, w_ref.at[:, pl.ds(0, n_half)], wl.at[0], right)
        b0 = rc(4, w_ref.at[:, pl.ds(0, n_half)], wr.at[0], left)
        a1 = rc(5, w_ref.at[:, pl.ds(n_half, n_half)], wl.at[1], right)
        b1 = rc(6, w_ref.at[:, pl.ds(n_half, n_half)], wr.at[1], left)
        for r in (c, d, e, a0, b0, a1, b1):
            r.start()

        xcp = pltpu.make_async_copy(
            x_ref.at[pl.ds(my * m_per, m_per), :], xg.at[my], copy_sems.at[0])
        xcp.start()
        xcp.wait()

        scale = sx_ref[0] * sw_ref[0]

        def dot(xs, ws):
            return lax.dot_general(
                xs, ws, (((1,), (0,)), ((), ())),
                preferred_element_type=jnp.float32,
            )

        def silu_inplace():
            y = acc[...] * scale
            acc[...] = y / (1.0 + jnp.exp(-jnp.clip(y, -60.0, 60.0)))

        acc[...] = dot(xg[my], w_ref[:, pl.ds(0, n_half)])
        a0.wait()
        b0.wait()
        f0 = rc(7, wl.at[0, pl.ds(0, k_half), :],
                wd.at[0, pl.ds(0, k_half), :], right)
        g0 = rc(8, wr.at[0, pl.ds(k_half, k_half), :],
                wd.at[0, pl.ds(k_half, k_half), :], left)
        f0.start()
        g0.start()
        c.wait()
        d.wait()
        acc[...] = acc[...] + dot(xg[left], wl[0])
        acc[...] = acc[...] + dot(xg[right], wr[0])
        e.wait()
        f0.wait()
        g0.wait()
        acc[...] = acc[...] + dot(xg[diag], wd[0])
        silu_inplace()
        out0 = pltpu.make_async_copy(
            acc, out_ref.at[:, pl.ds(0, n_half)], copy_sems.at[1])
        out0.start()

        a1.wait()
        b1.wait()
        f1 = rc(9, wl.at[1, pl.ds(0, k_half), :],
                wd.at[1, pl.ds(0, k_half), :], right)
        g1 = rc(10, wr.at[1, pl.ds(k_half, k_half), :],
                wd.at[1, pl.ds(k_half, k_half), :], left)
        f1.start()
        g1.start()
        out0.wait()
        acc[...] = dot(xg[my], w_ref[:, pl.ds(n_half, n_half)])
        acc[...] = acc[...] + dot(xg[left], wl[1])
        acc[...] = acc[...] + dot(xg[right], wr[1])
        f1.wait()
        g1.wait()
        acc[...] = acc[...] + dot(xg[diag], wd[1])
        silu_inplace()
        out1 = pltpu.make_async_copy(
            acc, out_ref.at[:, pl.ds(n_half, n_half)], copy_sems.at[2])
        out1.start()
        out1.wait()

    return pl.pallas_call(
        body,
        in_specs=[
            pl.BlockSpec(memory_space=pl.ANY),
            pl.BlockSpec(memory_space=pltpu.VMEM),
            pl.BlockSpec(memory_space=pltpu.SMEM),
            pl.BlockSpec(memory_space=pltpu.SMEM),
        ],
        out_specs=pl.BlockSpec(memory_space=pl.ANY),
        out_shape=jax.ShapeDtypeStruct((m_per, n_tot), jnp.float32),
        scratch_shapes=[
            pltpu.VMEM((N_DEV, m_per, k_loc), jnp.float8_e4m3fn),
            pltpu.VMEM((2, k_loc, n_half), jnp.float8_e4m3fn),
            pltpu.VMEM((2, k_loc, n_half), jnp.float8_e4m3fn),
            pltpu.VMEM((2, k_loc, n_half), jnp.float8_e4m3fn),
            pltpu.VMEM((m_per, n_half), jnp.float32),
            pltpu.SemaphoreType.DMA((11,)),
            pltpu.SemaphoreType.DMA((11,)),
            pltpu.SemaphoreType.DMA((3,)),
        ],
        compiler_params=pltpu.CompilerParams(
            collective_id=0,
            vmem_limit_bytes=60 * 1024 * 1024,
        ),
    )(x, w_mat, scale_x, scale_w)


# device time: 226154 ns/iter; 5.4175x vs baseline; 1.0466x over previous
import jax
import jax.numpy as jnp
from jax import lax
from jax.experimental import pallas as pl
from jax.experimental.pallas import tpu as pltpu

N_DEV = 4


def kernel(x, w_mat, scale_x, scale_w):
    m_tot, k_loc = x.shape
    _, n_tot = w_mat.shape
    m_per = m_tot // N_DEV
    kh = k_loc // 2
    nh = n_tot // 2

    x = x.astype(jnp.float8_e4m3fn)
    w_mat = w_mat.astype(jnp.float8_e4m3fn)

    def body(x_ref, w_ref, sx_ref, sw_ref, out_ref,
             xg, wl, wr, wd, acc, send_sems, recv_sems, copy_sems):
        my = lax.axis_index("i")
        left = lax.rem(my + N_DEV - 1, N_DEV)
        right = lax.rem(my + 1, N_DEV)
        diag = lax.rem(my + 2, N_DEV)

        barrier = pltpu.get_barrier_semaphore()
        for nbr in (left, right, diag):
            pl.semaphore_signal(
                barrier, inc=1, device_id=(nbr,),
                device_id_type=pl.DeviceIdType.MESH,
            )
        pl.semaphore_wait(barrier, 3)

        def rc(i, src, dst, dev):
            return pltpu.make_async_remote_copy(
                src_ref=src, dst_ref=dst,
                send_sem=send_sems.at[i], recv_sem=recv_sems.at[i],
                device_id=(dev,), device_id_type=pl.DeviceIdType.MESH,
            )

        def dot(xs, ws):
            return lax.dot_general(
                xs, ws, (((1,), (0,)), ((), ())),
                preferred_element_type=jnp.float32,
            )

        def silu_store(half):
            y = acc[...] * (sx_ref[0] * sw_ref[0])
            acc[...] = y / (1.0 + jnp.exp(-jnp.clip(y, -60.0, 60.0)))
            cp = pltpu.make_async_copy(
                acc, out_ref.at[:, pl.ds(half * nh, nh)], copy_sems.at[half])
            cp.start()
            return cp

        c = rc(0, x_ref.at[pl.ds(right * m_per, m_per), :], xg.at[my], right)
        d = rc(1, x_ref.at[pl.ds(left * m_per, m_per), :], xg.at[my], left)
        e = rc(2, x_ref.at[pl.ds(diag * m_per, m_per), :], xg.at[my], diag)
        a0t = rc(3, w_ref.at[pl.ds(0, kh), pl.ds(0, nh)],
                 wl.at[0, pl.ds(0, kh), :], right)
        a0b = rc(4, w_ref.at[pl.ds(kh, kh), pl.ds(0, nh)],
                 wl.at[0, pl.ds(kh, kh), :], right)
        b0b = rc(5, w_ref.at[pl.ds(kh, kh), pl.ds(0, nh)],
                 wr.at[0, pl.ds(kh, kh), :], left)
        b0t = rc(6, w_ref.at[pl.ds(0, kh), pl.ds(0, nh)],
                 wr.at[0, pl.ds(0, kh), :], left)
        for r in (c, d, e, a0t, a0b, b0b, b0t):
            r.start()

        xcp = pltpu.make_async_copy(
            x_ref.at[pl.ds(my * m_per, m_per), :], xg.at[my], copy_sems.at[2])
        xcp.start()
        xcp.wait()
        acc[...] = dot(xg[my], w_ref[:, pl.ds(0, nh)])
        c.wait()
        d.wait()

        a0t.wait()
        f0 = rc(7, wl.at[0, pl.ds(0, kh), :], wd.at[0, pl.ds(0, kh), :], right)
        f0.start()
        b0b.wait()
        g0 = rc(8, wr.at[0, pl.ds(kh, kh), :], wd.at[0, pl.ds(kh, kh), :], left)
        g0.start()
        a1t = rc(9, w_ref.at[pl.ds(0, kh), pl.ds(nh, nh)],
                 wl.at[1, pl.ds(0, kh), :], right)
        a1b = rc(10, w_ref.at[pl.ds(kh, kh), pl.ds(nh, nh)],
                 wl.at[1, pl.ds(kh, kh), :], right)
        b1b = rc(11, w_ref.at[pl.ds(kh, kh), pl.ds(nh, nh)],
                 wr.at[1, pl.ds(kh, kh), :], left)
        b1t = rc(12, w_ref.at[pl.ds(0, kh), pl.ds(nh, nh)],
                 wr.at[1, pl.ds(0, kh), :], left)
        for r in (a1t, a1b, b1b, b1t):
            r.start()

        acc[...] = acc[...] + dot(xg[left][:, 0:kh], wl[0, pl.ds(0, kh), :])
        acc[...] = acc[...] + dot(xg[right][:, kh:k_loc],
                                  wr[0, pl.ds(kh, kh), :])
        a0b.wait()
        acc[...] = acc[...] + dot(xg[left][:, kh:k_loc],
                                  wl[0, pl.ds(kh, kh), :])
        b0t.wait()
        acc[...] = acc[...] + dot(xg[right][:, 0:kh], wr[0, pl.ds(0, kh), :])
        e.wait()
        f0.wait()
        g0.wait()
        acc[...] = acc[...] + dot(xg[diag], wd[0])
        out0 = silu_store(0)

        a1t.wait()
        f1 = rc(13, wl.at[1, pl.ds(0, kh), :], wd.at[1, pl.ds(0, kh), :], right)
        f1.start()
        b1b.wait()
        g1 = rc(14, wr.at[1, pl.ds(kh, kh), :], wd.at[1, pl.ds(kh, kh), :], left)
        g1.start()
        out0.wait()
        acc[...] = dot(xg[my], w_ref[:, pl.ds(nh, nh)])
        acc[...] = acc[...] + dot(xg[left][:, 0:kh], wl[1, pl.ds(0, kh), :])
        acc[...] = acc[...] + dot(xg[right][:, kh:k_loc],
                                  wr[1, pl.ds(kh, kh), :])
        a1b.wait()
        acc[...] = acc[...] + dot(xg[left][:, kh:k_loc],
                                  wl[1, pl.ds(kh, kh), :])
        b1t.wait()
        acc[...] = acc[...] + dot(xg[right][:, 0:kh], wr[1, pl.ds(0, kh), :])
        f1.wait()
        g1.wait()
        acc[...] = acc[...] + dot(xg[diag], wd[1])
        out1 = silu_store(1)
        out1.wait()

    return pl.pallas_call(
        body,
        in_specs=[
            pl.BlockSpec(memory_space=pl.ANY),
            pl.BlockSpec(memory_space=pltpu.VMEM),
            pl.BlockSpec(memory_space=pltpu.SMEM),
            pl.BlockSpec(memory_space=pltpu.SMEM),
        ],
        out_specs=pl.BlockSpec(memory_space=pl.ANY),
        out_shape=jax.ShapeDtypeStruct((m_per, n_tot), jnp.float32),
        scratch_shapes=[
            pltpu.VMEM((N_DEV, m_per, k_loc), jnp.float8_e4m3fn),
            pltpu.VMEM((2, k_loc, nh), jnp.float8_e4m3fn),
            pltpu.VMEM((2, k_loc, nh), jnp.float8_e4m3fn),
            pltpu.VMEM((2, k_loc, nh), jnp.float8_e4m3fn),
            pltpu.VMEM((m_per, nh), jnp.float32),
            pltpu.SemaphoreType.DMA((15,)),
            pltpu.SemaphoreType.DMA((15,)),
            pltpu.SemaphoreType.DMA((3,)),
        ],
        compiler_params=pltpu.CompilerParams(
            collective_id=0,
            vmem_limit_bytes=60 * 1024 * 1024,
        ),
    )(x, w_mat, scale_x, scale_w)


# device time: 217211 ns/iter; 5.6405x vs baseline; 1.0412x over previous
import jax
import jax.numpy as jnp
from jax import lax
from jax.experimental import pallas as pl
from jax.experimental.pallas import tpu as pltpu

N_DEV = 4
NQ = 8


def kernel(x, w_mat, scale_x, scale_w):
    m_tot, k_loc = x.shape
    _, n_tot = w_mat.shape
    m_per = m_tot // N_DEV
    kh = k_loc // 2
    nq = n_tot // NQ

    x = x.astype(jnp.float8_e4m3fn)
    w_mat = w_mat.astype(jnp.float8_e4m3fn)

    def body(x_ref, w_ref, sx_ref, sw_ref, out_ref,
             xg, wl, wr, wd, send_sems, recv_sems, copy_sems):
        q = pl.program_id(0)
        my = lax.axis_index("i")
        left = lax.rem(my + N_DEV - 1, N_DEV)
        right = lax.rem(my + 1, N_DEV)
        diag = lax.rem(my + 2, N_DEV)

        def rc(i, src, dst, dev):
            return pltpu.make_async_remote_copy(
                src_ref=src, dst_ref=dst,
                send_sem=send_sems.at[i], recv_sem=recv_sems.at[i],
                device_id=(dev,), device_id_type=pl.DeviceIdType.MESH,
            )

        def desc_a_t(t):
            return rc(t, w_ref.at[pl.ds(0, kh), pl.ds(t * nq, nq)],
                      wl.at[t, pl.ds(0, kh), :], right)

        def desc_a_b(t):
            return rc(8 + t, w_ref.at[pl.ds(kh, kh), pl.ds(t * nq, nq)],
                      wl.at[t, pl.ds(kh, kh), :], right)

        def desc_b_b(t):
            return rc(16 + t, w_ref.at[pl.ds(kh, kh), pl.ds(t * nq, nq)],
                      wr.at[t, pl.ds(kh, kh), :], left)

        def desc_b_t(t):
            return rc(24 + t, w_ref.at[pl.ds(0, kh), pl.ds(t * nq, nq)],
                      wr.at[t, pl.ds(0, kh), :], left)

        def desc_f(t):
            return rc(32 + t, wl.at[t, pl.ds(0, kh), :],
                      wd.at[t, pl.ds(0, kh), :], right)

        def desc_g(t):
            return rc(40 + t, wr.at[t, pl.ds(kh, kh), :],
                      wd.at[t, pl.ds(kh, kh), :], left)

        def desc_x(i, src_dev):
            return rc(48 + i, x_ref.at[pl.ds(src_dev * m_per, m_per), :],
                      xg.at[my], src_dev)

        def start_p1(t):
            desc_a_t(t).start()
            desc_a_b(t).start()
            desc_b_b(t).start()
            desc_b_t(t).start()

        @pl.when(q == 0)
        def _():
            barrier = pltpu.get_barrier_semaphore()
            for nbr in (left, right, diag):
                pl.semaphore_signal(
                    barrier, inc=1, device_id=(nbr,),
                    device_id_type=pl.DeviceIdType.MESH,
                )
            pl.semaphore_wait(barrier, 3)
            desc_x(0, right).start()
            desc_x(1, left).start()
            desc_x(2, diag).start()
            start_p1(0)
            xcp = pltpu.make_async_copy(
                x_ref.at[pl.ds(my * m_per, m_per), :],
                xg.at[my], copy_sems.at[0])
            xcp.start()
            xcp.wait()

        desc_a_t(q).wait()
        desc_f(q).start()
        desc_b_b(q).wait()
        desc_g(q).start()

        @pl.when(q < NQ - 1)
        def _():
            start_p1(q + 1)

        def dot(xs, ws):
            return lax.dot_general(
                xs, ws, (((1,), (0,)), ((), ())),
                preferred_element_type=jnp.float32,
            )

        out_ref[...] = dot(xg[my], w_ref[:, pl.ds(q * nq, nq)])

        @pl.when(q == 0)
        def _():
            desc_x(0, right).wait()
            desc_x(1, left).wait()

        out_ref[...] = out_ref[...] + dot(xg[left][:, 0:kh],
                                          wl[q, pl.ds(0, kh), :])
        out_ref[...] = out_ref[...] + dot(xg[right][:, kh:k_loc],
                                          wr[q, pl.ds(kh, kh), :])
        desc_a_b(q).wait()
        out_ref[...] = out_ref[...] + dot(xg[left][:, kh:k_loc],
                                          wl[q, pl.ds(kh, kh), :])
        desc_b_t(q).wait()
        out_ref[...] = out_ref[...] + dot(xg[right][:, 0:kh],
                                          wr[q, pl.ds(0, kh), :])

        @pl.when(q == 0)
        def _():
            desc_x(2, diag).wait()

        desc_f(q).wait()
        desc_g(q).wait()
        out_ref[...] = out_ref[...] + dot(xg[diag], wd[q])

        y = out_ref[...] * (sx_ref[0] * sw_ref[0])
        out_ref[...] = y / (1.0 + jnp.exp(-jnp.clip(y, -60.0, 60.0)))

    return pl.pallas_call(
        body,
        grid=(NQ,),
        in_specs=[
            pl.BlockSpec(memory_space=pl.ANY),
            pl.BlockSpec((k_loc, n_tot), lambda q: (0, 0)),
            pl.BlockSpec(memory_space=pltpu.SMEM),
            pl.BlockSpec(memory_space=pltpu.SMEM),
        ],
        out_specs=pl.BlockSpec((m_per, nq), lambda q: (0, q)),
        out_shape=jax.ShapeDtypeStruct((m_per, n_tot), jnp.float32),
        scratch_shapes=[
            pltpu.VMEM((N_DEV, m_per, k_loc), jnp.float8_e4m3fn),
            pltpu.VMEM((NQ, k_loc, nq), jnp.float8_e4m3fn),
            pltpu.VMEM((NQ, k_loc, nq), jnp.float8_e4m3fn),
            pltpu.VMEM((NQ, k_loc, nq), jnp.float8_e4m3fn),
            pltpu.SemaphoreType.DMA((51,)),
            pltpu.SemaphoreType.DMA((51,)),
            pltpu.SemaphoreType.DMA((1,)),
        ],
        compiler_params=pltpu.CompilerParams(
            dimension_semantics=("arbitrary",),
            collective_id=0,
            vmem_limit_bytes=56 * 1024 * 1024,
        ),
    )(x, w_mat, scale_x, scale_w)


# device time: 216169 ns/iter; 5.6677x vs baseline; 1.0048x over previous
import jax
import jax.numpy as jnp
from jax import lax
from jax.experimental import pallas as pl
from jax.experimental.pallas import tpu as pltpu

N_DEV = 4
NQ = 16


def kernel(x, w_mat, scale_x, scale_w):
    m_tot, k_loc = x.shape
    _, n_tot = w_mat.shape
    m_per = m_tot // N_DEV
    kh = k_loc // 2
    nq = n_tot // NQ

    x = x.astype(jnp.float8_e4m3fn)
    w_mat = w_mat.astype(jnp.float8_e4m3fn)

    def body(x_ref, w_ref, sx_ref, sw_ref, out_ref,
             xg, wl, wr, wd, send_sems, recv_sems, copy_sems):
        q = pl.program_id(0)
        my = lax.axis_index("i")
        left = lax.rem(my + N_DEV - 1, N_DEV)
        right = lax.rem(my + 1, N_DEV)
        diag = lax.rem(my + 2, N_DEV)

        def rc(i, src, dst, dev):
            return pltpu.make_async_remote_copy(
                src_ref=src, dst_ref=dst,
                send_sem=send_sems.at[i], recv_sem=recv_sems.at[i],
                device_id=(dev,), device_id_type=pl.DeviceIdType.MESH,
            )

        def desc_a_t(t):
            return rc(t, w_ref.at[pl.ds(0, kh), pl.ds(t * nq, nq)],
                      wl.at[t, pl.ds(0, kh), :], right)

        def desc_a_b(t):
            return rc(NQ + t, w_ref.at[pl.ds(kh, kh), pl.ds(t * nq, nq)],
                      wl.at[t, pl.ds(kh, kh), :], right)

        def desc_b_b(t):
            return rc(2 * NQ + t, w_ref.at[pl.ds(kh, kh), pl.ds(t * nq, nq)],
                      wr.at[t, pl.ds(kh, kh), :], left)

        def desc_b_t(t):
            return rc(3 * NQ + t, w_ref.at[pl.ds(0, kh), pl.ds(t * nq, nq)],
                      wr.at[t, pl.ds(0, kh), :], left)

        def desc_f(t):
            return rc(4 * NQ + t, wl.at[t, pl.ds(0, kh), :],
                      wd.at[t, pl.ds(0, kh), :], right)

        def desc_g(t):
            return rc(5 * NQ + t, wr.at[t, pl.ds(kh, kh), :],
                      wd.at[t, pl.ds(kh, kh), :], left)

        def desc_x(i, src_dev):
            return rc(6 * NQ + i, x_ref.at[pl.ds(src_dev * m_per, m_per), :],
                      xg.at[my], src_dev)

        def start_p1(t):
            desc_a_t(t).start()
            desc_a_b(t).start()
            desc_b_b(t).start()
            desc_b_t(t).start()

        @pl.when(q == 0)
        def _():
            barrier = pltpu.get_barrier_semaphore()
            for nbr in (left, right, diag):
                pl.semaphore_signal(
                    barrier, inc=1, device_id=(nbr,),
                    device_id_type=pl.DeviceIdType.MESH,
                )
            pl.semaphore_wait(barrier, 3)
            start_p1(0)
            desc_x(0, right).start()
            desc_x(1, left).start()
            desc_x(2, diag).start()
            xcp = pltpu.make_async_copy(
                x_ref.at[pl.ds(my * m_per, m_per), :],
                xg.at[my], copy_sems.at[0])
            xcp.start()
            xcp.wait()

        desc_a_t(q).wait()
        desc_f(q).start()
        desc_b_b(q).wait()
        desc_g(q).start()

        @pl.when(q < NQ - 1)
        def _():
            start_p1(q + 1)

        def dot(xs, ws):
            return lax.dot_general(
                xs, ws, (((1,), (0,)), ((), ())),
                preferred_element_type=jnp.float32,
            )

        out_ref[...] = dot(xg[my], w_ref[:, pl.ds(q * nq, nq)])

        @pl.when(q == 0)
        def _():
            desc_x(0, right).wait()
            desc_x(1, left).wait()

        out_ref[...] = out_ref[...] + dot(xg[left][:, 0:kh],
                                          wl[q, pl.ds(0, kh), :])
        out_ref[...] = out_ref[...] + dot(xg[right][:, kh:k_loc],
                                          wr[q, pl.ds(kh, kh), :])
        desc_a_b(q).wait()
        out_ref[...] = out_ref[...] + dot(xg[left][:, kh:k_loc],
                                          wl[q, pl.ds(kh, kh), :])
        desc_b_t(q).wait()
        out_ref[...] = out_ref[...] + dot(xg[right][:, 0:kh],
                                          wr[q, pl.ds(0, kh), :])

        @pl.when(q == 0)
        def _():
            desc_x(2, diag).wait()

        desc_f(q).wait()
        desc_g(q).wait()
        out_ref[...] = out_ref[...] + dot(xg[diag], wd[q])

        y = out_ref[...] * (sx_ref[0] * sw_ref[0])
        out_ref[...] = y / (1.0 + jnp.exp(-jnp.clip(y, -60.0, 60.0)))

    return pl.pallas_call(
        body,
        grid=(NQ,),
        in_specs=[
            pl.BlockSpec(memory_space=pl.ANY),
            pl.BlockSpec((k_loc, n_tot), lambda q: (0, 0)),
            pl.BlockSpec(memory_space=pltpu.SMEM),
            pl.BlockSpec(memory_space=pltpu.SMEM),
        ],
        out_specs=pl.BlockSpec((m_per, nq), lambda q: (0, q)),
        out_shape=jax.ShapeDtypeStruct((m_per, n_tot), jnp.float32),
        scratch_shapes=[
            pltpu.VMEM((N_DEV, m_per, k_loc), jnp.float8_e4m3fn),
            pltpu.VMEM((NQ, k_loc, nq), jnp.float8_e4m3fn),
            pltpu.VMEM((NQ, k_loc, nq), jnp.float8_e4m3fn),
            pltpu.VMEM((NQ, k_loc, nq), jnp.float8_e4m3fn),
            pltpu.SemaphoreType.DMA((6 * NQ + 3,)),
            pltpu.SemaphoreType.DMA((6 * NQ + 3,)),
            pltpu.SemaphoreType.DMA((1,)),
        ],
        compiler_params=pltpu.CompilerParams(
            dimension_semantics=("arbitrary",),
            collective_id=0,
            vmem_limit_bytes=56 * 1024 * 1024,
        ),
    )(x, w_mat, scale_x, scale_w)


# device time: 191294 ns/iter; 6.4047x vs baseline; 1.1300x over previous
import jax
import jax.numpy as jnp
from jax import lax
from jax.experimental import pallas as pl
from jax.experimental.pallas import tpu as pltpu

N_DEV = 4
NQ = 16


def kernel(x, w_mat, scale_x, scale_w):
    m_tot, k_loc = x.shape
    _, n_tot = w_mat.shape
    m_per = m_tot // N_DEV
    kh = k_loc // 2
    nq = n_tot // NQ

    def body(x_ref, w_ref, sx_ref, sw_ref, out_ref,
             xg, wl, wr, wd, wf8, xf8, stage, send_sems, recv_sems, copy_sems):
        q = pl.program_id(0)
        my = lax.axis_index("i")
        left = lax.rem(my + N_DEV - 1, N_DEV)
        right = lax.rem(my + 1, N_DEV)
        diag = lax.rem(my + 2, N_DEV)

        def cvt_w(t, slot):
            cp = pltpu.make_async_copy(
                w_ref.at[:, pl.ds(t * nq, nq)],
                stage.at[slot, :, pl.ds(0, nq)], copy_sems.at[slot])
            cp.start()
            cp.wait()
            wf8[:, pl.ds(t * nq, nq)] = stage[
                slot, :, pl.ds(0, nq)].astype(jnp.float8_e4m3fn)

        def cvt_x(dev, slot):
            cp = pltpu.make_async_copy(
                x_ref.at[pl.ds(dev * m_per, m_per), :],
                stage.at[slot], copy_sems.at[slot])
            cp.start()
            cp.wait()
            xf8[pl.ds(dev * m_per, m_per), :] = stage[slot].astype(
                jnp.float8_e4m3fn)

        def rc(i, src, dst, dev):
            return pltpu.make_async_remote_copy(
                src_ref=src, dst_ref=dst,
                send_sem=send_sems.at[i], recv_sem=recv_sems.at[i],
                device_id=(dev,), device_id_type=pl.DeviceIdType.MESH,
            )

        def desc_a_t(t):
            return rc(t, wf8.at[pl.ds(0, kh), pl.ds(t * nq, nq)],
                      wl.at[t, pl.ds(0, kh), :], right)

        def desc_a_b(t):
            return rc(NQ + t, wf8.at[pl.ds(kh, kh), pl.ds(t * nq, nq)],
                      wl.at[t, pl.ds(kh, kh), :], right)

        def desc_b_b(t):
            return rc(2 * NQ + t, wf8.at[pl.ds(kh, kh), pl.ds(t * nq, nq)],
                      wr.at[t, pl.ds(kh, kh), :], left)

        def desc_b_t(t):
            return rc(3 * NQ + t, wf8.at[pl.ds(0, kh), pl.ds(t * nq, nq)],
                      wr.at[t, pl.ds(0, kh), :], left)

        def desc_f(t):
            return rc(4 * NQ + t, wl.at[t, pl.ds(0, kh), :],
                      wd.at[t, pl.ds(0, kh), :], right)

        def desc_g(t):
            return rc(5 * NQ + t, wr.at[t, pl.ds(kh, kh), :],
                      wd.at[t, pl.ds(kh, kh), :], left)

        def desc_x(i, src_dev):
            return rc(6 * NQ + i, xf8.at[pl.ds(src_dev * m_per, m_per), :],
                      xg.at[my], src_dev)

        def start_p1(t):
            desc_a_t(t).start()
            desc_a_b(t).start()
            desc_b_b(t).start()
            desc_b_t(t).start()

        @pl.when(q == 0)
        def _():
            cvt_w(0, 0)
            barrier = pltpu.get_barrier_semaphore()
            for nbr in (left, right, diag):
                pl.semaphore_signal(
                    barrier, inc=1, device_id=(nbr,),
                    device_id_type=pl.DeviceIdType.MESH,
                )
            pl.semaphore_wait(barrier, 3)
            start_p1(0)
            cvt_x(right, 0)
            desc_x(0, right).start()
            cvt_x(left, 1)
            desc_x(1, left).start()
            cvt_x(diag, 0)
            desc_x(2, diag).start()
            cvt_w(1, 1)
            cvt_x(my, 0)
            xcp = pltpu.make_async_copy(
                xf8.at[pl.ds(my * m_per, m_per), :],
                xg.at[my], copy_sems.at[0])
            xcp.start()
            xcp.wait()

        @pl.when(jnp.logical_and(q >= 1, q < NQ - 1))
        def _():
            cvt_w(q + 1, q % 2)

        desc_a_t(q).wait()
        desc_f(q).start()
        desc_b_b(q).wait()
        desc_g(q).start()

        @pl.when(q < NQ - 1)
        def _():
            start_p1(q + 1)

        def dot(xs, ws):
            return lax.dot_general(
                xs, ws, (((1,), (0,)), ((), ())),
                preferred_element_type=jnp.float32,
            )

        out_ref[...] = dot(xg[my], wf8[:, pl.ds(q * nq, nq)])

        @pl.when(q == 0)
        def _():
            desc_x(0, right).wait()
            desc_x(1, left).wait()

        out_ref[...] = out_ref[...] + dot(xg[left][:, 0:kh],
                                          wl[q, pl.ds(0, kh), :])
        out_ref[...] = out_ref[...] + dot(xg[right][:, kh:k_loc],
                                          wr[q, pl.ds(kh, kh), :])
        desc_a_b(q).wait()
        out_ref[...] = out_ref[...] + dot(xg[left][:, kh:k_loc],
                                          wl[q, pl.ds(kh, kh), :])
        desc_b_t(q).wait()
        out_ref[...] = out_ref[...] + dot(xg[right][:, 0:kh],
                                          wr[q, pl.ds(0, kh), :])

        @pl.when(q == 0)
        def _():
            desc_x(2, diag).wait()

        desc_f(q).wait()
        desc_g(q).wait()
        out_ref[...] = out_ref[...] + dot(xg[diag], wd[q])

        y = out_ref[...] * (sx_ref[0] * sw_ref[0])
        out_ref[...] = y / (1.0 + jnp.exp(-jnp.clip(y, -60.0, 60.0)))

    return pl.pallas_call(
        body,
        grid=(NQ,),
        in_specs=[
            pl.BlockSpec(memory_space=pl.ANY),
            pl.BlockSpec(memory_space=pl.ANY),
            pl.BlockSpec(memory_space=pltpu.SMEM),
            pl.BlockSpec(memory_space=pltpu.SMEM),
        ],
        out_specs=pl.BlockSpec((m_per, nq), lambda q: (0, q)),
        out_shape=jax.ShapeDtypeStruct((m_per, n_tot), jnp.float32),
        scratch_shapes=[
            pltpu.VMEM((N_DEV, m_per, k_loc), jnp.float8_e4m3fn),
            pltpu.VMEM((NQ, k_loc, nq), jnp.float8_e4m3fn),
            pltpu.VMEM((NQ, k_loc, nq), jnp.float8_e4m3fn),
            pltpu.VMEM((NQ, k_loc, nq), jnp.float8_e4m3fn),
            pltpu.VMEM((k_loc, n_tot), jnp.float8_e4m3fn),
            pltpu.VMEM((m_tot, k_loc), jnp.float8_e4m3fn),
            pltpu.VMEM((2, k_loc, k_loc), jnp.float32),
            pltpu.SemaphoreType.DMA((6 * NQ + 3,)),
            pltpu.SemaphoreType.DMA((6 * NQ + 3,)),
            pltpu.SemaphoreType.DMA((2,)),
        ],
        compiler_params=pltpu.CompilerParams(
            dimension_semantics=("arbitrary",),
            collective_id=0,
            vmem_limit_bytes=60 * 1024 * 1024,
        ),
    )(x, w_mat, scale_x, scale_w)
